# Initial kernel scaffold; baseline (speedup 1.0000x reference)
import jax
import jax.numpy as jnp
from jax.experimental import pallas as pl


def kernel(inputs, emb):
    raise NotImplementedError("write your pallas kernel here")


# speedup vs baseline: 2.4052x; 2.4052x over previous
import jax
import jax.numpy as jnp
from jax.experimental import pallas as pl

_NUM_EMB = 1024
_NUM_HEADS = 4
_DH = 256
_D = _NUM_HEADS * _DH
_COMMITMENT_COST = 0.25
_BN = 512


def _vq_kernel(x_ref, emb_ref, q_ref, codes_ref, loss_ref):
    i = pl.program_id(0)
    x = x_ref[...]
    q_parts = []
    for h in range(_NUM_HEADS):
        xh = x[:, h * _DH:(h + 1) * _DH]
        eh = emb_ref[h]
        xx = jnp.sum(xh * xh, axis=1, keepdims=True)
        ee = jnp.sum(eh * eh, axis=1)[None, :]
        ip = jax.lax.dot_general(xh, eh, (((1,), (1,)), ((), ())),
                                 preferred_element_type=jnp.float32)
        dist = -1.0 * (xx + ee - 2.0 * ip)
        m = jnp.max(dist, axis=1, keepdims=True)
        p = jnp.exp(dist - m)
        s = jnp.sum(p, axis=1, keepdims=True)
        probs = p / s
        pm = jnp.max(probs, axis=1, keepdims=True)
        lanes = jax.lax.broadcasted_iota(jnp.int32, probs.shape, 1)
        code = jnp.min(jnp.where(probs == pm, lanes, _NUM_EMB),
                       axis=1, keepdims=True)
        codes_ref[:, h:h + 1] = code
        qh = jax.lax.dot_general(probs, eh, (((1,), (0,)), ((), ())),
                                 preferred_element_type=jnp.float32)
        q_parts.append(qh)
    q = jnp.concatenate(q_parts, axis=1)
    q_ref[...] = x + (q - x)
    part = jnp.sum((q - x) ** 2)

    @pl.when(i == 0)
    def _init():
        loss_ref[...] = jnp.zeros_like(loss_ref)

    loss_ref[...] += jnp.full(loss_ref.shape, part, jnp.float32)


def kernel(inputs, emb):
    x = inputs[:, 0, :]
    n = x.shape[0]
    q, codes, loss_acc = pl.pallas_call(
        _vq_kernel,
        grid=(n // _BN,),
        in_specs=[
            pl.BlockSpec((_BN, _D), lambda i: (i, 0)),
            pl.BlockSpec((_NUM_HEADS, _NUM_EMB, _DH), lambda i: (0, 0, 0)),
        ],
        out_specs=[
            pl.BlockSpec((_BN, _D), lambda i: (i, 0)),
            pl.BlockSpec((_BN, _NUM_HEADS), lambda i: (i, 0)),
            pl.BlockSpec((1, 1, 128), lambda i: (0, 0, 0)),
        ],
        out_shape=[
            jax.ShapeDtypeStruct((n, _D), jnp.float32),
            jax.ShapeDtypeStruct((n, _NUM_HEADS), jnp.int32),
            jax.ShapeDtypeStruct((1, 1, 128), jnp.float32),
        ],
    )(x, emb)
    loss = loss_acc[0, 0, 0] * (_COMMITMENT_COST / (n * _D))
    return loss, q.reshape(inputs.shape), codes


# speedup vs baseline: 2.6760x; 1.1126x over previous
import jax
import jax.numpy as jnp
from jax.experimental import pallas as pl
from jax.experimental.pallas import tpu as pltpu

_NUM_EMB = 1024
_NUM_HEADS = 4
_DH = 256
_D = _NUM_HEADS * _DH
_COMMITMENT_COST = 0.25
_BN = 512


def _vq_kernel(x_ref, emb_ref, q_ref, codes_ref, loss_ref, ee_ref):
    i = pl.program_id(0)

    @pl.when(i == 0)
    def _init():
        for h in range(_NUM_HEADS):
            eh = emb_ref[h]
            ee_ref[h:h + 1, :] = jnp.sum(eh * eh, axis=1)[None, :]
        loss_ref[...] = jnp.zeros_like(loss_ref)

    x = x_ref[...]
    lanes = jax.lax.broadcasted_iota(jnp.int32, (_BN, _NUM_EMB), 1)
    loss_part = jnp.float32(0.0)
    for h in range(_NUM_HEADS):
        xh = x[:, h * _DH:(h + 1) * _DH]
        eh = emb_ref[h]
        ip2 = jax.lax.dot_general(xh + xh, eh, (((1,), (1,)), ((), ())),
                                  preferred_element_type=jnp.float32)
        logits = ip2 - ee_ref[h:h + 1, :]
        m = jnp.max(logits, axis=1, keepdims=True)
        code = jnp.min(jnp.where(logits == m, lanes, _NUM_EMB),
                       axis=1, keepdims=True)
        codes_ref[:, h:h + 1] = code
        p = jnp.exp(logits - m)
        s = jnp.sum(p, axis=1, keepdims=True)
        qh = jax.lax.dot_general(p, eh, (((1,), (0,)), ((), ())),
                                 preferred_element_type=jnp.float32) / s
        q_ref[:, h * _DH:(h + 1) * _DH] = qh
        dh = qh - xh
        loss_part += jnp.sum(dh * dh)

    loss_ref[...] += jnp.full(loss_ref.shape, loss_part, jnp.float32)


def kernel(inputs, emb):
    x = inputs[:, 0, :]
    n = x.shape[0]
    q, codes, loss_acc = pl.pallas_call(
        _vq_kernel,
        grid=(n // _BN,),
        in_specs=[
            pl.BlockSpec((_BN, _D), lambda i: (i, 0)),
            pl.BlockSpec((_NUM_HEADS, _NUM_EMB, _DH), lambda i: (0, 0, 0)),
        ],
        out_specs=[
            pl.BlockSpec((_BN, _D), lambda i: (i, 0)),
            pl.BlockSpec((_BN, _NUM_HEADS), lambda i: (i, 0)),
            pl.BlockSpec((1, 1, 128), lambda i: (0, 0, 0)),
        ],
        out_shape=[
            jax.ShapeDtypeStruct((n, _D), jnp.float32),
            jax.ShapeDtypeStruct((n, _NUM_HEADS), jnp.int32),
            jax.ShapeDtypeStruct((1, 1, 128), jnp.float32),
        ],
        scratch_shapes=[pltpu.VMEM((_NUM_HEADS, _NUM_EMB), jnp.float32)],
    )(x, emb)
    loss = loss_acc[0, 0, 0] * (_COMMITMENT_COST / (n * _D))
    return loss, q.reshape(inputs.shape), codes
